# baseline (speedup 1.0000x reference)
import functools

import jax
import jax.numpy as jnp
from jax import lax
from jax.experimental import pallas as pl
from jax.experimental.pallas import tpu as pltpu
from jax.experimental.pallas import tpu_sc as plsc

N = 10000
C = 128
K = 32
L = 16
NC, NS = 2, 16
NPAD = 10240
GS = 8
GK = GS * K
FAST = 0
CHF = 640
CHS = 0
NF = NS * CHF
GF = CHF // GS
GSL = 0
CW = C // 2
CG = CW // L
TJ = 1024


def _tc_body(x_ref, w_ref, a_ref, h_ref, s_ref):
    xb = x_ref[...]
    hb = lax.dot_general(xb, w_ref[...],
                         (((0,), (1,)), ((), ())),
                         preferred_element_type=jnp.float32)
    hb = jnp.maximum(hb, 0.0)
    h_ref[...] = hb.astype(jnp.bfloat16)
    s_ref[...] = lax.dot_general(a_ref[...], hb,
                                 (((0,), (1,)), ((), ())),
                                 preferred_element_type=jnp.float32)


_tc_call = pl.pallas_call(
    _tc_body,
    grid=(NPAD // TJ,),
    in_specs=[
        pl.BlockSpec((C, TJ), lambda j: (0, j)),
        pl.BlockSpec((C, C), lambda j: (0, 0)),
        pl.BlockSpec((C, 8), lambda j: (0, 0)),
    ],
    out_specs=[
        pl.BlockSpec((TJ, C), lambda j: (j, 0)),
        pl.BlockSpec((8, TJ), lambda j: (0, j)),
    ],
    out_shape=[
        jax.ShapeDtypeStruct((NPAD, C), jnp.bfloat16),
        jax.ShapeDtypeStruct((8, NPAD), jnp.float32),
    ],
)


def _compute_group(g, base, rows_v, self_v, out_v, s1_v, s2_v, e0_v, e1_v,
                   w_v, ws_v):
    lanes = lax.iota(jnp.int32, L)
    lane0 = lanes == 0

    def node_body(nig, carry):
        ib = (g * GS + nig) * K
        r32 = nig * K
        nid = base + g * GS + nig

        def leaky(v):
            return jnp.where(v >= 0.0, v, v * 0.2)

        def logits(lo):
            idxv = ib + lo + lanes
            i1 = plsc.load_gather(e1_v, [idxv])
            i0 = plsc.load_gather(e0_v, [idxv])
            return leaky(plsc.load_gather(s1_v, [i1])
                         + plsc.load_gather(s2_v, [i0]))

        al_a = logits(0)
        al_b = logits(L)
        nv = jnp.broadcast_to(nid, (L,))
        sv = leaky(plsc.load_gather(s1_v, [nv]) + plsc.load_gather(s2_v, [nv]))
        al_c = jnp.where(lane0, sv, -1e30)
        m = jnp.max(jnp.maximum(jnp.maximum(al_a, al_b), al_c))
        ea = jnp.exp(al_a - m)
        eb = jnp.exp(al_b - m)
        ec = jnp.exp(al_c - m)
        ssum = jnp.sum(ea + eb + ec)
        inv = jnp.ones((L,), jnp.float32) / jnp.broadcast_to(ssum, (L,))
        plsc.store_scatter(w_v, [r32 + lanes], ea * inv)
        plsc.store_scatter(w_v, [r32 + L + lanes], eb * inv)
        nr = jnp.broadcast_to(nig, (L,))
        plsc.store_scatter(ws_v, [nr], ec * inv, mask=lane0)

        def term(acc_e, acc_o, src_v, row, wk):
            ne, no = [], []
            for c in range(CG):
                pair = plsc.load_gather(src_v, [row, lanes + c * L])
                bf = plsc.bitcast(pair, jnp.bfloat16)
                ev, od = plsc.unpack(bf, format=plsc.PackFormat.INTERLEAVED)
                ne.append(acc_e[c] + wk * ev.astype(jnp.float32))
                no.append(acc_o[c] + wk * od.astype(jnp.float32))
            return ne, no

        def kbody(k, accs):
            acc_e, acc_o = accs
            rk = jnp.broadcast_to(r32 + k, (L,))
            wk = plsc.load_gather(w_v, [rk])
            ne, no = term(acc_e, acc_o, rows_v, rk, wk)
            return tuple(ne), tuple(no)

        zeros = tuple(jnp.zeros((L,), jnp.float32) for _ in range(CG))
        acc_e, acc_o = lax.fori_loop(0, K, kbody, (zeros, zeros))
        wself = plsc.load_gather(ws_v, [nr])
        acc_e, acc_o = term(acc_e, acc_o, self_v, nr, wself)
        for c in range(CG):
            plsc.store_scatter(out_v, [nr, 2 * lanes + c * 2 * L], acc_e[c])
            plsc.store_scatter(out_v, [nr, 2 * lanes + c * 2 * L + 1],
                               acc_o[c])
        return carry

    lax.fori_loop(0, GS, node_body, 0)


_sc_mesh = plsc.VectorSubcoreMesh(core_axis_name="c", subcore_axis_name="s")


@functools.partial(
    pl.kernel,
    mesh=_sc_mesh,
    out_type=jax.ShapeDtypeStruct((NPAD, C), jnp.float32),
    scratch_types=[
        pltpu.VMEM((NPAD,), jnp.float32),
        pltpu.VMEM((NPAD,), jnp.float32),
        pltpu.VMEM((CHF * K,), jnp.int32),
        pltpu.VMEM((CHF * K,), jnp.int32),
        pltpu.VMEM((GK, CW), jnp.int32),
        pltpu.VMEM((GK, CW), jnp.int32),
        pltpu.VMEM((GS, CW), jnp.int32),
        pltpu.VMEM((GS, CW), jnp.int32),
        pltpu.VMEM((GK,), jnp.float32),
        pltpu.VMEM((L,), jnp.float32),
        pltpu.VMEM((GS, C), jnp.float32),
        pltpu.VMEM((GS, C), jnp.float32),
        pltpu.SemaphoreType.DMA,
        pltpu.SemaphoreType.DMA,
        pltpu.SemaphoreType.DMA,
        pltpu.SemaphoreType.DMA,
    ],
    compiler_params=pltpu.CompilerParams(
        needs_layout_passes=False, use_tc_tiling_on_sc=False),
)
def _sc_gat(h_hbm, s1_hbm, s2_hbm, e0_hbm, e1_hbm, out_hbm,
            s1_v, s2_v, e0_v, e1_v, rows_a, rows_b, self_a, self_b,
            w_v, ws_v, out_a, out_b, gsem_a, gsem_b, osem_a, osem_b):
    cidx = lax.axis_index("c")
    sidx = lax.axis_index("s")
    is_fast = cidx == FAST
    base = jnp.where(is_fast, sidx * CHF, 0)
    gcount = jnp.where(is_fast, GF, GSL)

    ebase = pl.multiple_of(base * K, 8)

    def g_copy(g, rows, selfr, sem):
        nb = pltpu.make_async_copy(
            h_hbm.at[e0_v.at[pl.ds(pl.multiple_of(g * GK, 8), GK)]],
            rows, sem)
        sf = pltpu.make_async_copy(
            h_hbm.at[pl.ds(pl.multiple_of(base + g * GS, 8), GS)],
            selfr, sem)
        return nb, sf

    def g_start(g, rows, selfr, sem):
        nb, sf = g_copy(g, rows, selfr, sem)
        nb.start()
        sf.start()

    def g_wait(g, rows, selfr, sem):
        nb, sf = g_copy(g, rows, selfr, sem)
        nb.wait()
        sf.wait()

    def o_copy(g, outb, sem):
        return pltpu.make_async_copy(
            outb, out_hbm.at[pl.ds(base + g * GS, GS)], sem)

    @pl.when(is_fast)
    def _():
        pltpu.sync_copy(s1_hbm, s1_v)
        pltpu.sync_copy(s2_hbm, s2_v)
        pltpu.sync_copy(e0_hbm.at[pl.ds(ebase, CHF * K)], e0_v)
        pltpu.sync_copy(e1_hbm.at[pl.ds(ebase, CHF * K)], e1_v)
        g_start(0, rows_a, self_a, gsem_a)

    def outer(t, carry):
        ga = 2 * t
        gb = ga + 1
        g_start(gb, rows_b, self_b, gsem_b)
        g_wait(ga, rows_a, self_a, gsem_a)

        @pl.when(t > 0)
        def _():
            o_copy(ga, out_a, osem_a).wait()

        _compute_group(ga, base, rows_a, self_a, out_a, s1_v, s2_v,
                       e0_v, e1_v, w_v, ws_v)
        o_copy(ga, out_a, osem_a).start()

        @pl.when(gb + 1 < gcount)
        def _():
            g_start(gb + 1, rows_a, self_a, gsem_a)

        g_wait(gb, rows_b, self_b, gsem_b)

        @pl.when(t > 0)
        def _():
            o_copy(gb, out_b, osem_b).wait()

        _compute_group(gb, base, rows_b, self_b, out_b, s1_v, s2_v,
                       e0_v, e1_v, w_v, ws_v)
        o_copy(gb, out_b, osem_b).start()
        return carry

    lax.fori_loop(0, gcount // 2, outer, 0)

    @pl.when(is_fast)
    def _():
        o_copy(gcount - 2, out_a, osem_a).wait()
        o_copy(gcount - 1, out_b, osem_b).wait()


def kernel(x, edge_index, W, att, b):
    att_i = att[0, :C, 0, 0]
    att_j = att[0, C:, 0, 0]
    att2 = jnp.zeros((C, 8), jnp.float32)
    att2 = att2.at[:, 0].set(att_i).at[:, 1].set(att_j)

    ef = jnp.pad(edge_index[:, 0].reshape(2, -1), ((0, 0), (0, (NPAD - N) * K)))

    h_t, s = _tc_call(x[0, :, :, 0], W, att2)
    h_i32 = lax.bitcast_convert_type(h_t.reshape(NPAD, CW, 2),
                                     jnp.int32)
    out_t = _sc_gat(h_i32, s[0], s[1], ef[0], ef[1])
    out = out_t[:N].T.reshape(1, C, N, 1)
    return out + b


# speedup vs baseline: 1.3933x; 1.3933x over previous
import functools

import jax
import jax.numpy as jnp
from jax import lax
from jax.experimental import pallas as pl
from jax.experimental.pallas import tpu as pltpu
from jax.experimental.pallas import tpu_sc as plsc

N = 10000
C = 128
K = 32
L = 16
NC, NS = 2, 16
NPAD = 10240
GS = 8
GK = GS * K
FAST = 0
CHF = 576
CHS = 64
NF = NS * CHF
GF = CHF // GS
GSL = CHS // GS
CW = C // 2
CG = CW // L
TJ = 1024


def _tc_body(x_ref, w_ref, a_ref, h_ref, s_ref):
    xb = x_ref[...]
    hb = lax.dot_general(xb, w_ref[...],
                         (((0,), (1,)), ((), ())),
                         preferred_element_type=jnp.float32)
    hb = jnp.maximum(hb, 0.0)
    h_ref[...] = hb.astype(jnp.bfloat16)
    s_ref[...] = lax.dot_general(a_ref[...], hb,
                                 (((0,), (1,)), ((), ())),
                                 preferred_element_type=jnp.float32)


_tc_call = pl.pallas_call(
    _tc_body,
    grid=(NPAD // TJ,),
    in_specs=[
        pl.BlockSpec((C, TJ), lambda j: (0, j)),
        pl.BlockSpec((C, C), lambda j: (0, 0)),
        pl.BlockSpec((C, 8), lambda j: (0, 0)),
    ],
    out_specs=[
        pl.BlockSpec((TJ, C), lambda j: (j, 0)),
        pl.BlockSpec((8, TJ), lambda j: (0, j)),
    ],
    out_shape=[
        jax.ShapeDtypeStruct((NPAD, C), jnp.bfloat16),
        jax.ShapeDtypeStruct((8, NPAD), jnp.float32),
    ],
)


def _compute_group(g, base, rows_v, self_v, out_v, s1_v, s2_v, e0_v, e1_v,
                   w_v, ws_v):
    lanes = lax.iota(jnp.int32, L)
    lane0 = lanes == 0

    def node_body(nig, carry):
        ib = (g * GS + nig) * K
        r32 = nig * K
        nid = base + g * GS + nig

        def leaky(v):
            return jnp.where(v >= 0.0, v, v * 0.2)

        def logits(lo):
            idxv = ib + lo + lanes
            i1 = plsc.load_gather(e1_v, [idxv])
            i0 = plsc.load_gather(e0_v, [idxv])
            return leaky(plsc.load_gather(s1_v, [i1])
                         + plsc.load_gather(s2_v, [i0]))

        al_a = logits(0)
        al_b = logits(L)
        nv = jnp.broadcast_to(nid, (L,))
        sv = leaky(plsc.load_gather(s1_v, [nv]) + plsc.load_gather(s2_v, [nv]))
        al_c = jnp.where(lane0, sv, -1e30)
        m = jnp.max(jnp.maximum(jnp.maximum(al_a, al_b), al_c))
        ea = jnp.exp(al_a - m)
        eb = jnp.exp(al_b - m)
        ec = jnp.exp(al_c - m)
        ssum = jnp.sum(ea + eb + ec)
        inv = jnp.ones((L,), jnp.float32) / jnp.broadcast_to(ssum, (L,))
        plsc.store_scatter(w_v, [r32 + lanes], ea * inv)
        plsc.store_scatter(w_v, [r32 + L + lanes], eb * inv)
        nr = jnp.broadcast_to(nig, (L,))
        plsc.store_scatter(ws_v, [nr], ec * inv, mask=lane0)

        def term(acc_e, acc_o, src_v, row, wk):
            ne, no = [], []
            for c in range(CG):
                pair = plsc.load_gather(src_v, [row, lanes + c * L])
                bf = plsc.bitcast(pair, jnp.bfloat16)
                ev, od = plsc.unpack(bf, format=plsc.PackFormat.INTERLEAVED)
                ne.append(acc_e[c] + wk * ev.astype(jnp.float32))
                no.append(acc_o[c] + wk * od.astype(jnp.float32))
            return ne, no

        def kbody(k, accs):
            acc_e, acc_o = accs
            rk = jnp.broadcast_to(r32 + k, (L,))
            wk = plsc.load_gather(w_v, [rk])
            ne, no = term(acc_e, acc_o, rows_v, rk, wk)
            return tuple(ne), tuple(no)

        zeros = tuple(jnp.zeros((L,), jnp.float32) for _ in range(CG))
        acc_e, acc_o = lax.fori_loop(0, K, kbody, (zeros, zeros))
        wself = plsc.load_gather(ws_v, [nr])
        acc_e, acc_o = term(acc_e, acc_o, self_v, nr, wself)
        for c in range(CG):
            plsc.store_scatter(out_v, [nr, 2 * lanes + c * 2 * L], acc_e[c])
            plsc.store_scatter(out_v, [nr, 2 * lanes + c * 2 * L + 1],
                               acc_o[c])
        return carry

    lax.fori_loop(0, GS, node_body, 0)


_sc_mesh = plsc.VectorSubcoreMesh(core_axis_name="c", subcore_axis_name="s")


@functools.partial(
    pl.kernel,
    mesh=_sc_mesh,
    out_type=jax.ShapeDtypeStruct((NPAD, C), jnp.float32),
    scratch_types=[
        pltpu.VMEM((NPAD,), jnp.float32),
        pltpu.VMEM((NPAD,), jnp.float32),
        pltpu.VMEM((CHF * K,), jnp.int32),
        pltpu.VMEM((CHF * K,), jnp.int32),
        pltpu.VMEM((GK, CW), jnp.int32),
        pltpu.VMEM((GK, CW), jnp.int32),
        pltpu.VMEM((GS, CW), jnp.int32),
        pltpu.VMEM((GS, CW), jnp.int32),
        pltpu.VMEM((GK,), jnp.float32),
        pltpu.VMEM((L,), jnp.float32),
        pltpu.VMEM((GS, C), jnp.float32),
        pltpu.VMEM((GS, C), jnp.float32),
        pltpu.SemaphoreType.DMA,
        pltpu.SemaphoreType.DMA,
        pltpu.SemaphoreType.DMA,
        pltpu.SemaphoreType.DMA,
    ],
    compiler_params=pltpu.CompilerParams(
        needs_layout_passes=False, use_tc_tiling_on_sc=False),
)
def _sc_gat(h_hbm, s1_hbm, s2_hbm, e0_hbm, e1_hbm, out_hbm,
            s1_v, s2_v, e0_v, e1_v, rows_a, rows_b, self_a, self_b,
            w_v, ws_v, out_a, out_b, gsem_a, gsem_b, osem_a, osem_b):
    cidx = lax.axis_index("c")
    sidx = lax.axis_index("s")
    is_fast = cidx == FAST
    base = jnp.where(is_fast, sidx * CHF, NF + sidx * CHS)
    gcount = jnp.where(is_fast, GF, GSL)

    pltpu.sync_copy(s1_hbm, s1_v)
    pltpu.sync_copy(s2_hbm, s2_v)
    ebase = pl.multiple_of(base * K, 8)

    @pl.when(is_fast)
    def _():
        pltpu.sync_copy(e0_hbm.at[pl.ds(ebase, CHF * K)], e0_v)
        pltpu.sync_copy(e1_hbm.at[pl.ds(ebase, CHF * K)], e1_v)

    @pl.when(jnp.logical_not(is_fast))
    def _():
        pltpu.sync_copy(e0_hbm.at[pl.ds(ebase, CHS * K)],
                        e0_v.at[pl.ds(0, CHS * K)])
        pltpu.sync_copy(e1_hbm.at[pl.ds(ebase, CHS * K)],
                        e1_v.at[pl.ds(0, CHS * K)])

    def g_copy(g, rows, selfr, sem):
        nb = pltpu.make_async_copy(
            h_hbm.at[e0_v.at[pl.ds(pl.multiple_of(g * GK, 8), GK)]],
            rows, sem)
        sf = pltpu.make_async_copy(
            h_hbm.at[pl.ds(pl.multiple_of(base + g * GS, 8), GS)],
            selfr, sem)
        return nb, sf

    def g_start(g, rows, selfr, sem):
        nb, sf = g_copy(g, rows, selfr, sem)
        nb.start()
        sf.start()

    def g_wait(g, rows, selfr, sem):
        nb, sf = g_copy(g, rows, selfr, sem)
        nb.wait()
        sf.wait()

    def o_copy(g, outb, sem):
        return pltpu.make_async_copy(
            outb, out_hbm.at[pl.ds(base + g * GS, GS)], sem)

    g_start(0, rows_a, self_a, gsem_a)

    def outer(t, carry):
        ga = 2 * t
        gb = ga + 1
        g_start(gb, rows_b, self_b, gsem_b)
        g_wait(ga, rows_a, self_a, gsem_a)

        @pl.when(t > 0)
        def _():
            o_copy(ga, out_a, osem_a).wait()

        _compute_group(ga, base, rows_a, self_a, out_a, s1_v, s2_v,
                       e0_v, e1_v, w_v, ws_v)
        o_copy(ga, out_a, osem_a).start()

        @pl.when(gb + 1 < gcount)
        def _():
            g_start(gb + 1, rows_a, self_a, gsem_a)

        g_wait(gb, rows_b, self_b, gsem_b)

        @pl.when(t > 0)
        def _():
            o_copy(gb, out_b, osem_b).wait()

        _compute_group(gb, base, rows_b, self_b, out_b, s1_v, s2_v,
                       e0_v, e1_v, w_v, ws_v)
        o_copy(gb, out_b, osem_b).start()
        return carry

    lax.fori_loop(0, gcount // 2, outer, 0)
    o_copy(gcount - 2, out_a, osem_a).wait()
    o_copy(gcount - 1, out_b, osem_b).wait()


def kernel(x, edge_index, W, att, b):
    att_i = att[0, :C, 0, 0]
    att_j = att[0, C:, 0, 0]
    att2 = jnp.zeros((C, 8), jnp.float32)
    att2 = att2.at[:, 0].set(att_i).at[:, 1].set(att_j)

    ef = jnp.pad(edge_index[:, 0].reshape(2, -1), ((0, 0), (0, (NPAD - N) * K)))

    h_t, s = _tc_call(x[0, :, :, 0], W, att2)
    h_i32 = lax.bitcast_convert_type(h_t.reshape(NPAD, CW, 2),
                                     jnp.int32)
    out_t = _sc_gat(h_i32, s[0], s[1], ef[0], ef[1])
    out = out_t[:N].T.reshape(1, C, N, 1)
    return out + b


# speedup vs baseline: 1.4062x; 1.0092x over previous
import functools

import jax
import jax.numpy as jnp
from jax import lax
from jax.experimental import pallas as pl
from jax.experimental.pallas import tpu as pltpu
from jax.experimental.pallas import tpu_sc as plsc

N = 10000
C = 128
K = 32
L = 16
NC, NS = 2, 16
NPAD = 10240
GS = 8
GK = GS * K
FAST = 0
CHF = 608
CHS = 32
NF = NS * CHF
GF = CHF // GS
GSL = CHS // GS
CW = C // 2
CG = CW // L
TJ = 1024


def _tc_body(x_ref, w_ref, a_ref, h_ref, s_ref):
    xb = x_ref[...]
    hb = lax.dot_general(xb, w_ref[...],
                         (((0,), (1,)), ((), ())),
                         preferred_element_type=jnp.float32)
    hb = jnp.maximum(hb, 0.0)
    h_ref[...] = hb.astype(jnp.bfloat16)
    s_ref[...] = lax.dot_general(a_ref[...], hb,
                                 (((0,), (1,)), ((), ())),
                                 preferred_element_type=jnp.float32)


_tc_call = pl.pallas_call(
    _tc_body,
    grid=(NPAD // TJ,),
    in_specs=[
        pl.BlockSpec((C, TJ), lambda j: (0, j)),
        pl.BlockSpec((C, C), lambda j: (0, 0)),
        pl.BlockSpec((C, 8), lambda j: (0, 0)),
    ],
    out_specs=[
        pl.BlockSpec((TJ, C), lambda j: (j, 0)),
        pl.BlockSpec((8, TJ), lambda j: (0, j)),
    ],
    out_shape=[
        jax.ShapeDtypeStruct((NPAD, C), jnp.bfloat16),
        jax.ShapeDtypeStruct((8, NPAD), jnp.float32),
    ],
)


def _compute_group(g, base, rows_v, self_v, out_v, s1_v, s2_v, e0_v, e1_v,
                   w_v, ws_v):
    lanes = lax.iota(jnp.int32, L)
    lane0 = lanes == 0

    def node_body(nig, carry):
        ib = (g * GS + nig) * K
        r32 = nig * K
        nid = base + g * GS + nig

        def leaky(v):
            return jnp.where(v >= 0.0, v, v * 0.2)

        def logits(lo):
            idxv = ib + lo + lanes
            i1 = plsc.load_gather(e1_v, [idxv])
            i0 = plsc.load_gather(e0_v, [idxv])
            return leaky(plsc.load_gather(s1_v, [i1])
                         + plsc.load_gather(s2_v, [i0]))

        al_a = logits(0)
        al_b = logits(L)
        nv = jnp.broadcast_to(nid, (L,))
        sv = leaky(plsc.load_gather(s1_v, [nv]) + plsc.load_gather(s2_v, [nv]))
        al_c = jnp.where(lane0, sv, -1e30)
        m = jnp.max(jnp.maximum(jnp.maximum(al_a, al_b), al_c))
        ea = jnp.exp(al_a - m)
        eb = jnp.exp(al_b - m)
        ec = jnp.exp(al_c - m)
        ssum = jnp.sum(ea + eb + ec)
        inv = jnp.ones((L,), jnp.float32) / jnp.broadcast_to(ssum, (L,))
        plsc.store_scatter(w_v, [r32 + lanes], ea * inv)
        plsc.store_scatter(w_v, [r32 + L + lanes], eb * inv)
        nr = jnp.broadcast_to(nig, (L,))
        plsc.store_scatter(ws_v, [nr], ec * inv, mask=lane0)

        def term(acc_e, acc_o, src_v, row, wk):
            ne, no = [], []
            for c in range(CG):
                pair = plsc.load_gather(src_v, [row, lanes + c * L])
                bf = plsc.bitcast(pair, jnp.bfloat16)
                ev, od = plsc.unpack(bf, format=plsc.PackFormat.INTERLEAVED)
                ne.append(acc_e[c] + wk * ev.astype(jnp.float32))
                no.append(acc_o[c] + wk * od.astype(jnp.float32))
            return ne, no

        def kbody(k, accs):
            acc_e, acc_o = accs
            rk = jnp.broadcast_to(r32 + k, (L,))
            wk = plsc.load_gather(w_v, [rk])
            ne, no = term(acc_e, acc_o, rows_v, rk, wk)
            return tuple(ne), tuple(no)

        zeros = tuple(jnp.zeros((L,), jnp.float32) for _ in range(CG))
        acc_e, acc_o = lax.fori_loop(0, K, kbody, (zeros, zeros))
        wself = plsc.load_gather(ws_v, [nr])
        acc_e, acc_o = term(acc_e, acc_o, self_v, nr, wself)
        for c in range(CG):
            plsc.store_scatter(out_v, [nr, 2 * lanes + c * 2 * L], acc_e[c])
            plsc.store_scatter(out_v, [nr, 2 * lanes + c * 2 * L + 1],
                               acc_o[c])
        return carry

    lax.fori_loop(0, GS, node_body, 0)


_sc_mesh = plsc.VectorSubcoreMesh(core_axis_name="c", subcore_axis_name="s")


@functools.partial(
    pl.kernel,
    mesh=_sc_mesh,
    out_type=jax.ShapeDtypeStruct((NPAD, C), jnp.float32),
    scratch_types=[
        pltpu.VMEM((NPAD,), jnp.float32),
        pltpu.VMEM((NPAD,), jnp.float32),
        pltpu.VMEM((CHF * K,), jnp.int32),
        pltpu.VMEM((CHF * K,), jnp.int32),
        pltpu.VMEM((GK, CW), jnp.int32),
        pltpu.VMEM((GK, CW), jnp.int32),
        pltpu.VMEM((GS, CW), jnp.int32),
        pltpu.VMEM((GS, CW), jnp.int32),
        pltpu.VMEM((GK,), jnp.float32),
        pltpu.VMEM((L,), jnp.float32),
        pltpu.VMEM((GS, C), jnp.float32),
        pltpu.VMEM((GS, C), jnp.float32),
        pltpu.SemaphoreType.DMA,
        pltpu.SemaphoreType.DMA,
        pltpu.SemaphoreType.DMA,
        pltpu.SemaphoreType.DMA,
    ],
    compiler_params=pltpu.CompilerParams(
        needs_layout_passes=False, use_tc_tiling_on_sc=False),
)
def _sc_gat(h_hbm, s1_hbm, s2_hbm, e0_hbm, e1_hbm, out_hbm,
            s1_v, s2_v, e0_v, e1_v, rows_a, rows_b, self_a, self_b,
            w_v, ws_v, out_a, out_b, gsem_a, gsem_b, osem_a, osem_b):
    cidx = lax.axis_index("c")
    sidx = lax.axis_index("s")
    is_fast = cidx == FAST
    base = jnp.where(is_fast, sidx * CHF, NF + sidx * CHS)
    gcount = jnp.where(is_fast, GF, GSL)

    pltpu.sync_copy(s1_hbm, s1_v)
    pltpu.sync_copy(s2_hbm, s2_v)
    ebase = pl.multiple_of(base * K, 8)

    @pl.when(is_fast)
    def _():
        pltpu.sync_copy(e0_hbm.at[pl.ds(ebase, CHF * K)], e0_v)
        pltpu.sync_copy(e1_hbm.at[pl.ds(ebase, CHF * K)], e1_v)

    @pl.when(jnp.logical_not(is_fast))
    def _():
        pltpu.sync_copy(e0_hbm.at[pl.ds(ebase, CHS * K)],
                        e0_v.at[pl.ds(0, CHS * K)])
        pltpu.sync_copy(e1_hbm.at[pl.ds(ebase, CHS * K)],
                        e1_v.at[pl.ds(0, CHS * K)])

    def g_copy(g, rows, selfr, sem):
        nb = pltpu.make_async_copy(
            h_hbm.at[e0_v.at[pl.ds(pl.multiple_of(g * GK, 8), GK)]],
            rows, sem)
        sf = pltpu.make_async_copy(
            h_hbm.at[pl.ds(pl.multiple_of(base + g * GS, 8), GS)],
            selfr, sem)
        return nb, sf

    def g_start(g, rows, selfr, sem):
        nb, sf = g_copy(g, rows, selfr, sem)
        nb.start()
        sf.start()

    def g_wait(g, rows, selfr, sem):
        nb, sf = g_copy(g, rows, selfr, sem)
        nb.wait()
        sf.wait()

    def o_copy(g, outb, sem):
        return pltpu.make_async_copy(
            outb, out_hbm.at[pl.ds(base + g * GS, GS)], sem)

    g_start(0, rows_a, self_a, gsem_a)

    def outer(t, carry):
        ga = 2 * t
        gb = ga + 1
        g_start(gb, rows_b, self_b, gsem_b)
        g_wait(ga, rows_a, self_a, gsem_a)

        @pl.when(t > 0)
        def _():
            o_copy(ga, out_a, osem_a).wait()

        _compute_group(ga, base, rows_a, self_a, out_a, s1_v, s2_v,
                       e0_v, e1_v, w_v, ws_v)
        o_copy(ga, out_a, osem_a).start()

        @pl.when(gb + 1 < gcount)
        def _():
            g_start(gb + 1, rows_a, self_a, gsem_a)

        g_wait(gb, rows_b, self_b, gsem_b)

        @pl.when(t > 0)
        def _():
            o_copy(gb, out_b, osem_b).wait()

        _compute_group(gb, base, rows_b, self_b, out_b, s1_v, s2_v,
                       e0_v, e1_v, w_v, ws_v)
        o_copy(gb, out_b, osem_b).start()
        return carry

    lax.fori_loop(0, gcount // 2, outer, 0)
    o_copy(gcount - 2, out_a, osem_a).wait()
    o_copy(gcount - 1, out_b, osem_b).wait()


def kernel(x, edge_index, W, att, b):
    att_i = att[0, :C, 0, 0]
    att_j = att[0, C:, 0, 0]
    att2 = jnp.zeros((C, 8), jnp.float32)
    att2 = att2.at[:, 0].set(att_i).at[:, 1].set(att_j)

    ef = jnp.pad(edge_index[:, 0].reshape(2, -1), ((0, 0), (0, (NPAD - N) * K)))

    h_t, s = _tc_call(x[0, :, :, 0], W, att2)
    h_i32 = lax.bitcast_convert_type(h_t.reshape(NPAD, CW, 2),
                                     jnp.int32)
    out_t = _sc_gat(h_i32, s[0], s[1], ef[0], ef[1])
    out = out_t[:N].T.reshape(1, C, N, 1)
    return out + b
